# Initial kernel scaffold; baseline (speedup 1.0000x reference)
import jax
import jax.numpy as jnp
from jax.experimental import pallas as pl


def kernel(node_feat, edge_feat, degs, pool_index, pool_value, n2p_index, n2p_value, e2p_index, e2p_value, params):
    raise NotImplementedError("write your pallas kernel here")


# speedup vs baseline: 6.6896x; 6.6896x over previous
import functools

import jax
import jax.numpy as jnp
from jax import lax
from jax.experimental import pallas as pl
from jax.experimental.pallas import tpu as pltpu
from jax.experimental.pallas import tpu_sc as plsc

_N = 50000
_E = 800000
_M = 50000
_L = 16
_HID = 32
_NL = 4
_NBOND = 4
_NATOM = 28
_NTASK = 1

_NC = 2
_NS = 16
_NW = _NC * _NS

_B = 512
_KJ = _B // 128
_CN = 49
_CP = 7

_BR = 2048
_NBLK = 25
_RPAD = _BR * _NBLK
_TROWS = _RPAD * _L
_MP = 50048
_SL = _MP // _NS


def _embed_body(nf_ref, tbl_ref, o_ref):
    nf = nf_ref[0, 0, :]
    oh = (nf[:, None] == lax.broadcasted_iota(jnp.int32, (_BR, 32), 1))
    o_ref[...] = jnp.dot(oh.astype(jnp.float32), tbl_ref[...],
                         preferred_element_type=jnp.float32)


def _embed(nf3, atom_pad):
    return pl.pallas_call(
        _embed_body,
        grid=(_NBLK,),
        in_specs=[pl.BlockSpec((1, 1, _BR), lambda i: (i, 0, 0)),
                  pl.BlockSpec((32, _HID), lambda i: (0, 0))],
        out_specs=pl.BlockSpec((_BR, _HID), lambda i: (i, 0)),
        out_shape=jax.ShapeDtypeStruct((_RPAD, _HID), jnp.float32),
    )(nf3, atom_pad)


def _mm_body(x_ref, w_ref, o_ref):
    o_ref[...] = jnp.dot(x_ref[...], w_ref[...],
                         preferred_element_type=jnp.float32)


def _matmul(x, w):
    return pl.pallas_call(
        _mm_body,
        grid=(_NBLK,),
        in_specs=[pl.BlockSpec((_BR, _HID), lambda i: (i, 0)),
                  pl.BlockSpec((_HID, _L * _HID), lambda i: (0, 0))],
        out_specs=pl.BlockSpec((_BR, _L * _HID), lambda i: (i, 0)),
        out_shape=jax.ShapeDtypeStruct((_RPAD, _L * _HID), jnp.float32),
    )(x, w)


def _fd_body(dg_ref, w0_ref, b0_ref, w1_ref, b1_ref, o_ref):
    x = dg_ref[0, 0, :]
    t = jnp.maximum(x[:, None] * w0_ref[0, 0, :][None, :]
                    + b0_ref[0, 0, :][None, :], 0.0)
    o_ref[0] = jnp.dot(t, w1_ref[0], preferred_element_type=jnp.float32) \
        + b1_ref[0]


def _fd_all(degs3, w0, b0, w1, b1):
    return pl.pallas_call(
        _fd_body,
        grid=(_NL, _NBLK),
        in_specs=[pl.BlockSpec((1, 1, _BR), lambda l, i: (i, 0, 0)),
                  pl.BlockSpec((1, 1, 2 * _HID), lambda l, i: (l, 0, 0)),
                  pl.BlockSpec((1, 1, 2 * _HID), lambda l, i: (l, 0, 0)),
                  pl.BlockSpec((1, 2 * _HID, _HID), lambda l, i: (l, 0, 0)),
                  pl.BlockSpec((1, 1, _HID), lambda l, i: (l, 0, 0))],
        out_specs=pl.BlockSpec((1, _BR, _HID), lambda l, i: (l, i, 0)),
        out_shape=jax.ShapeDtypeStruct((_NL, _RPAD, _HID), jnp.float32),
    )(degs3, w0, b0, w1, b1)


def _comb1_body(a_ref, b_ref, o_ref):
    o_ref[...] = jnp.maximum(a_ref[0] + a_ref[1] + b_ref[0:1, :], 0.0)


def _combine_relu(acc_parts, bias_pad):
    return pl.pallas_call(
        _comb1_body,
        grid=(_NBLK,),
        in_specs=[pl.BlockSpec((2, 2000, _HID), lambda i: (0, i, 0)),
                  pl.BlockSpec((8, _HID), lambda i: (0, 0))],
        out_specs=pl.BlockSpec((2000, _HID), lambda i: (i, 0)),
        out_shape=jax.ShapeDtypeStruct((_M, _HID), jnp.float32),
    )(acc_parts, bias_pad)


def _comb2_body(a_ref, f_ref, o_ref):
    o_ref[...] = (a_ref[0] + a_ref[1]) * f_ref[...]


def _combine_mul(hn_parts, fd_l):
    return pl.pallas_call(
        _comb2_body,
        grid=(_NBLK,),
        in_specs=[pl.BlockSpec((2, 2000, _HID), lambda i: (0, i, 0)),
                  pl.BlockSpec((2000, _HID), lambda i: (i, 0))],
        out_specs=pl.BlockSpec((2000, _HID), lambda i: (i, 0)),
        out_shape=jax.ShapeDtypeStruct((_N, _HID), jnp.float32),
    )(hn_parts, fd_l)


def _head_body(h_ref, fw_ref, fb_ref, o_ref, acc_ref):
    i = pl.program_id(0)

    @pl.when(i == 0)
    def _():
        acc_ref[...] = jnp.zeros_like(acc_ref)

    acc_ref[0:1, :] += jnp.sum(h_ref[...], axis=0, keepdims=True)

    @pl.when(i == _NBLK - 1)
    def _():
        s = jnp.sum(acc_ref[0:1, :] * fw_ref[0:1, :]) / _N + fb_ref[0, 0]
        o_ref[...] = jnp.full((8, 128), s, jnp.float32)


def _head(h, fw_pad, fb_pad):
    return pl.pallas_call(
        _head_body,
        grid=(_NBLK,),
        in_specs=[pl.BlockSpec((2000, _HID), lambda i: (i, 0)),
                  pl.BlockSpec((8, _HID), lambda i: (0, 0)),
                  pl.BlockSpec((8, _HID), lambda i: (0, 0))],
        out_specs=pl.BlockSpec((8, 128), lambda i: (0, 0)),
        out_shape=jax.ShapeDtypeStruct((8, 128), jnp.float32),
        scratch_shapes=[pltpu.VMEM((8, _HID), jnp.float32)],
    )(h, fw_pad, fb_pad)


def _mesh():
    return plsc.VectorSubcoreMesh(core_axis_name="c", subcore_axis_name="s",
                                  num_cores=_NC, num_subcores=_NS)


_GDN = lax.GatherDimensionNumbers(offset_dims=(), collapsed_slice_dims=(0,),
                                  start_index_map=(0,))


def _bcast_lane(vv, t):
    idx = jnp.broadcast_to(jnp.int32(t), (16, 1))
    return lax.gather(vv, idx, _GDN, (1,),
                      mode=lax.GatherScatterMode.PROMISE_IN_BOUNDS)


def _scale_rows(gath, vbuf):
    def group(g, carry):
        base = g * 16
        vv = vbuf[pl.ds(base, 16)]
        for t in range(16):
            i = base + t
            bv = _bcast_lane(vv, t)
            gath[i, pl.ds(0, 16)] = gath[i, pl.ds(0, 16)] * bv
            gath[i, pl.ds(16, 16)] = gath[i, pl.ds(16, 16)] * bv
        return carry
    lax.fori_loop(0, _B // 16, group, 0)


def _sc_stage1_body(table, nr, nc_, nv, er, ec, ev, efeat, zeros, out,
                    rbuf, cbuf, vbuf, ebuf, gbuf, dbuf, gath, acc, sem):
    cid = lax.axis_index("c")
    sid = lax.axis_index("s")
    wid = cid * _NS + sid
    pltpu.sync_copy(zeros.at[pl.ds(sid * _SL, _SL)],
                    acc.at[pl.ds(sid * _SL, _SL)])
    plsc.subcore_barrier()

    def do_chunk(rows_hbm, cols_hbm, vals_hbm, c, is_edge):
        pltpu.sync_copy(rows_hbm.at[wid, c], rbuf)
        pltpu.sync_copy(cols_hbm.at[wid, c], cbuf)
        pltpu.sync_copy(vals_hbm.at[wid, c], vbuf)
        if is_edge:
            cps = [pltpu.async_copy(efeat.at[cbuf.at[j]], ebuf.at[j], sem)
                   for j in range(_KJ)]
            for cp in cps:
                cp.wait()
        for j in range(_KJ):
            for t in range(8):
                sl = pl.ds(t * 16, 16)
                rv = rbuf[j, sl]
                a = jnp.bitwise_and(rv, _L - 1)
                if is_edge:
                    base = ebuf[j, sl] + _N
                else:
                    base = cbuf[j, sl]
                gbuf[j, sl] = base * _L + a
                dbuf[j, sl] = jnp.right_shift(rv, 4)
        cps = [pltpu.async_copy(table.at[gbuf.at[j]],
                                gath.at[pl.ds(j * 128, 128)], sem)
               for j in range(_KJ)]
        for cp in cps:
            cp.wait()
        _scale_rows(gath, vbuf)
        for j in range(_KJ):
            pltpu.sync_copy(gath.at[pl.ds(j * 128, 128)],
                            acc.at[dbuf.at[j]], add=True)

    def n_body(c, carry):
        do_chunk(nr, nc_, nv, c, False)
        return carry

    def e_body(c, carry):
        do_chunk(er, ec, ev, c, True)
        return carry

    lax.fori_loop(0, _CN, n_body, 0)
    lax.fori_loop(0, _CN, e_body, 0)
    plsc.subcore_barrier()
    pltpu.sync_copy(acc.at[pl.ds(sid * _SL, _SL)],
                    out.at[cid, pl.ds(sid * _SL, _SL)])


def _sc_stage1(table, nr, nc_, nv, er, ec, ev, efeat, zeros):
    k = pl.kernel(
        _sc_stage1_body,
        out_type=jax.ShapeDtypeStruct((_NC, _MP, _HID), jnp.float32),
        mesh=_mesh(),
        compiler_params=pltpu.CompilerParams(use_tc_tiling_on_sc=False),
        scratch_types=[
            pltpu.VMEM((_KJ, 128), jnp.int32),
            pltpu.VMEM((_KJ, 128), jnp.int32),
            pltpu.VMEM((_B,), jnp.float32),
            pltpu.VMEM((_KJ, 128), jnp.int32),
            pltpu.VMEM((_KJ, 128), jnp.int32),
            pltpu.VMEM((_KJ, 128), jnp.int32),
            pltpu.VMEM((_B, _HID), jnp.float32),
            pltpu.VMEM_SHARED((_MP, _HID), jnp.float32),
            pltpu.SemaphoreType.DMA,
        ],
    )
    return k(table, nr, nc_, nv, er, ec, ev, efeat, zeros)


def _sc_pool_body(mid, pr, pc, pv, zeros, out,
                  rbuf, cbuf, vbuf, gath, acc, sem):
    cid = lax.axis_index("c")
    sid = lax.axis_index("s")
    wid = cid * _NS + sid
    pltpu.sync_copy(zeros.at[pl.ds(sid * _SL, _SL)],
                    acc.at[pl.ds(sid * _SL, _SL)])
    plsc.subcore_barrier()

    def body(c, carry):
        pltpu.sync_copy(pr.at[wid, c], rbuf)
        pltpu.sync_copy(pc.at[wid, c], cbuf)
        pltpu.sync_copy(pv.at[wid, c], vbuf)
        cps = [pltpu.async_copy(mid.at[cbuf.at[j]],
                                gath.at[pl.ds(j * 128, 128)], sem)
               for j in range(_KJ)]
        for cp in cps:
            cp.wait()
        _scale_rows(gath, vbuf)
        for j in range(_KJ):
            pltpu.sync_copy(gath.at[pl.ds(j * 128, 128)],
                            acc.at[rbuf.at[j]], add=True)
        return carry

    lax.fori_loop(0, _CP, body, 0)
    plsc.subcore_barrier()
    pltpu.sync_copy(acc.at[pl.ds(sid * _SL, _SL)],
                    out.at[cid, pl.ds(sid * _SL, _SL)])


def _sc_pool(mid, pr, pc, pv, zeros):
    k = pl.kernel(
        _sc_pool_body,
        out_type=jax.ShapeDtypeStruct((_NC, _MP, _HID), jnp.float32),
        mesh=_mesh(),
        compiler_params=pltpu.CompilerParams(use_tc_tiling_on_sc=False),
        scratch_types=[
            pltpu.VMEM((_KJ, 128), jnp.int32),
            pltpu.VMEM((_KJ, 128), jnp.int32),
            pltpu.VMEM((_B,), jnp.float32),
            pltpu.VMEM((_B, _HID), jnp.float32),
            pltpu.VMEM_SHARED((_MP, _HID), jnp.float32),
            pltpu.SemaphoreType.DMA,
        ],
    )
    return k(mid, pr, pc, pv, zeros)


def _pack_nnz(index, value, chunks, dst_mod, col_mod, row_scale):
    total = _NW * chunks * _B
    npad = total - value.shape[0]
    ar = jnp.arange(npad, dtype=jnp.int32)
    rows = jnp.concatenate([index[0].astype(jnp.int32),
                            (ar % dst_mod) * row_scale])
    cols = jnp.concatenate([index[1].astype(jnp.int32), ar % col_mod])
    vals = jnp.concatenate([value.astype(jnp.float32),
                            jnp.zeros((npad,), jnp.float32)])
    rows = rows.reshape(_NW, chunks, _KJ, 128)
    cols = cols.reshape(_NW, chunks, _KJ, 128)
    vals = vals.reshape(_NW, chunks, _B)
    return rows, cols, vals


def kernel(node_feat, edge_feat, degs, pool_index, pool_value,
           n2p_index, n2p_value, e2p_index, e2p_value, params):
    f32 = jnp.float32
    w = params['weights']
    w2 = w.transpose(0, 1, 3, 2).reshape(_NL, _HID, _L * _HID)
    atom_pad = jnp.pad(params['atom_table'].astype(f32),
                       ((0, 32 - _NATOM), (0, 0)))
    bond = params['bond_tables'].astype(f32)
    bias_pad = jnp.broadcast_to(params['bias'].astype(f32), (_NL, 8, _HID))
    d0w = params['deg0_w'].astype(f32).transpose(0, 2, 1)
    d0b = params['deg0_b'].astype(f32).reshape(_NL, 1, 2 * _HID)
    d1w = params['deg1_w'].astype(f32).transpose(0, 2, 1)
    d1b = params['deg1_b'].astype(f32).reshape(_NL, 1, _HID)
    fw_pad = jnp.pad(params['final_w'].astype(f32), ((0, 7), (0, 0)))
    fb_pad = jnp.broadcast_to(params['final_b'].astype(f32).reshape(1, 1),
                              (8, _HID))

    nf3 = jnp.pad(node_feat.astype(jnp.int32),
                  (0, _RPAD - _N)).reshape(_NBLK, 1, _BR)
    degs3 = jnp.pad(degs.astype(f32), (0, _RPAD - _N)).reshape(_NBLK, 1, _BR)
    efeat = edge_feat.astype(jnp.int32)

    nr, nc_, nv = _pack_nnz(n2p_index, n2p_value, _CN, _M, _N, _L)
    er, ec, ev = _pack_nnz(e2p_index, e2p_value, _CN, _M, _E, _L)
    pr, pc, pv = _pack_nnz(pool_index, pool_value, _CP, _N, _M, 1)
    zeros = jnp.zeros((_MP, _HID), f32)

    fd = _fd_all(degs3, d0w, d0b, d1w, d1b)
    h = _embed(nf3, atom_pad)[:_N]

    pad_rows = jnp.zeros((_RPAD - _N - _NBOND, _HID), f32)
    for l in range(_NL):
        hcat = jnp.concatenate([h, bond[l], pad_rows], axis=0)
        x = _matmul(hcat, w2[l])
        table = x.reshape(_TROWS, _HID)
        acc_parts = _sc_stage1(table, nr, nc_, nv, er, ec, ev, efeat, zeros)
        mid = _combine_relu(acc_parts, bias_pad[l])
        hn_parts = _sc_pool(mid, pr, pc, pv, zeros)
        h = _combine_mul(hn_parts, fd[l, :_N])

    out = _head(h, fw_pad, fb_pad)
    return out[0:1, 0:1]
